# baseline (speedup 1.0000x reference)
import jax
import jax.numpy as jnp
from jax.experimental import pallas as pl

NUM_POS_FEATS = 128
BATCH_BLOCK = 8


def _pos_body(row_ref, col_ref, out_ref):
    bb = out_ref.shape[0]
    f = NUM_POS_FEATS
    h = row_ref.shape[0]
    w = col_ref.shape[0]
    colT = col_ref[...].T
    rowT = row_ref[...].T
    xp = jnp.tile(colT, (1, h))
    yp = jnp.repeat(rowT, w, axis=1)
    plane = jnp.concatenate([xp, yp], axis=0)
    out_ref[...] = jnp.broadcast_to(plane[None], (bb, 2 * f, h * w))


def kernel(mask, row_embed, col_embed):
    b, h, w = mask.shape
    f = NUM_POS_FEATS
    bb = BATCH_BLOCK
    out = pl.pallas_call(
        _pos_body,
        grid=(b // bb,),
        in_specs=[
            pl.BlockSpec((h, f), lambda i: (0, 0)),
            pl.BlockSpec((w, f), lambda i: (0, 0)),
        ],
        out_specs=pl.BlockSpec((bb, 2 * f, h * w), lambda i: (i, 0, 0)),
        out_shape=jax.ShapeDtypeStruct((b, 2 * f, h * w), jnp.float32),
    )(row_embed[:h], col_embed[:w])
    return out.reshape(b, 2 * f, h, w)


# speedup vs baseline: 1.0041x; 1.0041x over previous
import jax
import jax.numpy as jnp
from jax.experimental import pallas as pl
from jax.experimental.pallas import tpu as pltpu

NUM_POS_FEATS = 128
REP = 4
NSEM = 8


def _pos_body(row_ref, col_ref, out_ref, stage_ref, sems):
    f = NUM_POS_FEATS
    h = row_ref.shape[0]
    w = col_ref.shape[0]
    b = out_ref.shape[0]
    colT = col_ref[...].T
    rowT = row_ref[...].T
    xp = jnp.tile(colT, (1, h))
    yp = jnp.repeat(rowT, w, axis=1)
    plane = jnp.concatenate([xp, yp], axis=0)
    stage_ref[...] = jnp.broadcast_to(plane[None], (REP, 2 * f, h * w))
    ncopies = b // REP
    copies = [
        pltpu.make_async_copy(
            stage_ref,
            out_ref.at[pl.ds(i * REP, REP)],
            sems.at[i % NSEM],
        )
        for i in range(ncopies)
    ]
    for c in copies:
        c.start()
    for c in copies:
        c.wait()


def kernel(mask, row_embed, col_embed):
    b, h, w = mask.shape
    f = NUM_POS_FEATS
    out = pl.pallas_call(
        _pos_body,
        in_specs=[
            pl.BlockSpec((h, f), lambda: (0, 0)),
            pl.BlockSpec((w, f), lambda: (0, 0)),
        ],
        out_specs=pl.BlockSpec(memory_space=pltpu.MemorySpace.HBM),
        out_shape=jax.ShapeDtypeStruct((b, 2 * f, h * w), jnp.float32),
        scratch_shapes=[
            pltpu.VMEM((REP, 2 * f, h * w), jnp.float32),
            pltpu.SemaphoreType.DMA((NSEM,)),
        ],
    )(row_embed[:h], col_embed[:w])
    return out.reshape(b, 2 * f, h, w)
